# baseline (speedup 1.0000x reference)
import functools

import jax
import jax.numpy as jnp
from jax import lax
from jax.experimental import pallas as pl
from jax.experimental.pallas import tpu as pltpu
from jax.experimental.pallas import tpu_sc as plsc

N_TABLE = 100000
TOTAL = 12800000
NUM_WORKERS = 32
PER_WORKER = TOTAL // NUM_WORKERS
CHUNK = 4000
NUM_CHUNKS = PER_WORKER // CHUNK
NBUF = 3
T_PIECES = 5
T_PIECE = N_TABLE // T_PIECES

_mesh = plsc.VectorSubcoreMesh(core_axis_name="c", subcore_axis_name="s")


@functools.partial(
    pl.kernel,
    mesh=_mesh,
    out_type=jax.ShapeDtypeStruct((TOTAL,), jnp.int32),
    scratch_types=(
        [pltpu.VMEM((N_TABLE,), jnp.int32)]
        + [pltpu.VMEM((CHUNK,), jnp.int32) for _ in range(2 * NBUF)]
        + [pltpu.SemaphoreType.DMA for _ in range(2 * NBUF + 1)]
        + [pltpu.VMEM_SHARED((N_TABLE,), jnp.int32)]
    ),
    compiler_params=pltpu.CompilerParams(needs_layout_passes=False),
)
def _gather_kernel(idx_hbm, table_hbm, out_hbm, table_v, *rest):
    idx_bufs = rest[:NBUF]
    out_bufs = rest[NBUF:2 * NBUF]
    in_sems = rest[2 * NBUF:3 * NBUF]
    out_sems = rest[3 * NBUF:4 * NBUF]
    sem_t = rest[4 * NBUF]
    spmem_t = rest[4 * NBUF + 1]
    wid = lax.axis_index("s") * 2 + lax.axis_index("c")
    base = wid * PER_WORKER

    def start_in(c, b):
        pltpu.async_copy(
            idx_hbm.at[pl.ds(base + c * CHUNK, CHUNK)], idx_bufs[b], in_sems[b])

    def wait_in(b):
        pltpu.make_async_copy(
            idx_hbm.at[pl.ds(0, CHUNK)], idx_bufs[b], in_sems[b]).wait()

    def start_out(c, b):
        pltpu.async_copy(
            out_bufs[b], out_hbm.at[pl.ds(base + c * CHUNK, CHUNK)], out_sems[b])

    def wait_out(b):
        pltpu.make_async_copy(
            out_bufs[b], out_hbm.at[pl.ds(0, CHUNK)], out_sems[b]).wait()

    def gather(b):
        idx_v = idx_bufs[b]
        out_v = out_bufs[b]

        @plsc.parallel_loop(0, CHUNK, 16, unroll=10)
        def step(si):
            iv = idx_v[pl.ds(si, 16)]
            out_v[pl.ds(si, 16)] = plsc.load_gather(table_v, [iv])

    for b in range(NBUF):
        start_in(b, b)

    sid = lax.axis_index("s")

    @pl.when(sid < 10)
    def _():
        off = sid * (N_TABLE // 10)
        pltpu.async_copy(
            table_hbm.at[pl.ds(off, N_TABLE // 10)],
            table_v.at[pl.ds(off, N_TABLE // 10)], sem_t).wait()
        pltpu.async_copy(
            table_v.at[pl.ds(off, N_TABLE // 10)],
            spmem_t.at[pl.ds(off, N_TABLE // 10)], sem_t).wait()

    plsc.subcore_barrier()
    pltpu.async_copy(spmem_t, table_v, sem_t).wait()

    def body(g, carry):
        for b in range(NBUF):
            c = NBUF * g + b
            wait_in(b)

            @pl.when(c >= NBUF)
            def _():
                wait_out(b)

            gather(b)
            start_out(c, b)

            @pl.when(c + NBUF < NUM_CHUNKS)
            def _():
                start_in(c + NBUF, b)
        return carry

    lax.fori_loop(0, NUM_CHUNKS // NBUF, body, 0)

    for r in range(NUM_CHUNKS % NBUF):
        c = (NUM_CHUNKS // NBUF) * NBUF + r
        wait_in(r)
        wait_out(r)
        gather(r)
        start_out(c, r)

    for b in range(NBUF):
        wait_out(b)


def kernel(raw_atom_index_array, inverse_real_atoms):
    flat_idx = (
        raw_atom_index_array.reshape(50000, 128, 2)
        .transpose(0, 2, 1)
        .reshape(-1)
    )
    out = _gather_kernel(flat_idx, inverse_real_atoms)
    return (
        out.reshape(50000, 2, 128)
        .transpose(0, 2, 1)
        .reshape(6400000, 2)
    )
